# Initial kernel scaffold; baseline (speedup 1.0000x reference)
import jax
import jax.numpy as jnp
from jax.experimental import pallas as pl


def kernel(x, edge_index, W1, b1, W2, b2):
    raise NotImplementedError("write your pallas kernel here")


# speedup vs baseline: 4.9680x; 4.9680x over previous
import functools

import jax
import jax.numpy as jnp
from jax import lax
from jax.experimental import pallas as pl
from jax.experimental.pallas import tpu as pltpu
from jax.experimental.pallas import tpu_sc as plsc

N = 10000
E = 320000
D = 128
H = 128

NC = 2
NS = 16
NW = NC * NS
L = 16

CH = 128
NCHUNK = E // CH
CH_BASE = NCHUNK // NW
CH_REM = NCHUNK % NW

NPAD = 10112
ROWS_PER_TILE = NPAD // NS
DEG_STRIPE = 640
DEGP = DEG_STRIPE * NS

_mesh = plsc.VectorSubcoreMesh(
    core_axis_name="c", subcore_axis_name="s", num_cores=NC, num_subcores=NS
)


def _worker_chunks(wid):
    cnt = CH_BASE + jnp.where(wid < CH_REM, 1, 0)
    start = wid * CH_BASE + jnp.minimum(wid, CH_REM)
    return start, cnt


@functools.partial(
    pl.kernel,
    out_type=jax.ShapeDtypeStruct((NC * DEGP,), jnp.float32),
    mesh=_mesh,
    compiler_params=pltpu.CompilerParams(needs_layout_passes=False),
    scratch_types=[
        pltpu.VMEM((1, CH), jnp.int32),
        pltpu.VMEM((CH,), jnp.float32),
        pltpu.VMEM_SHARED((DEGP,), jnp.float32),
    ],
)
def _deg_kernel(dst_hbm, zeros1_hbm, out_hbm, didx, ones_v, sh_deg):
    c = lax.axis_index("c")
    s = lax.axis_index("s")
    wid = s * NC + c
    one = jnp.full((L,), 1.0, dtype=jnp.float32)
    for k in range(CH // L):
        ones_v[pl.ds(k * L, L)] = one
    pltpu.sync_copy(
        zeros1_hbm.at[pl.ds(0, DEG_STRIPE)],
        sh_deg.at[pl.ds(s * DEG_STRIPE, DEG_STRIPE)],
    )
    plsc.subcore_barrier()

    start, cnt = _worker_chunks(wid)

    def body(t, _):
        base = (start + t) * CH
        pltpu.sync_copy(dst_hbm.at[pl.ds(base, CH)], didx.at[0])
        pltpu.sync_copy(ones_v, sh_deg.at[didx.at[0]], add=True)
        return 0

    lax.fori_loop(0, cnt, body, 0)
    plsc.subcore_barrier()
    pltpu.sync_copy(
        sh_deg.at[pl.ds(s * DEG_STRIPE, DEG_STRIPE)],
        out_hbm.at[pl.ds(c * DEGP + s * DEG_STRIPE, DEG_STRIPE)],
    )


@functools.partial(
    pl.kernel,
    out_type=jax.ShapeDtypeStruct((NC, NPAD, H), jnp.float32),
    mesh=_mesh,
    compiler_params=pltpu.CompilerParams(needs_layout_passes=False),
    scratch_types=[
        pltpu.VMEM((1, CH), jnp.int32),
        pltpu.VMEM((1, CH), jnp.int32),
        pltpu.VMEM((CH, H), jnp.float32),
        pltpu.VMEM_SHARED((NPAD, H), jnp.float32),
        pltpu.SemaphoreType.DMA,
    ],
)
def _agg_kernel(hs_hbm, src_hbm, dst_hbm, zeros2_hbm, out_hbm,
                sidx, didx, rows, sh_acc, sem):
    c = lax.axis_index("c")
    s = lax.axis_index("s")
    wid = s * NC + c
    r0 = s * ROWS_PER_TILE
    pltpu.sync_copy(
        zeros2_hbm.at[pl.ds(0, ROWS_PER_TILE)],
        sh_acc.at[pl.ds(r0, ROWS_PER_TILE)],
    )
    plsc.subcore_barrier()

    start, cnt = _worker_chunks(wid)

    def body(t, _):
        base = (start + t) * CH
        pltpu.sync_copy(src_hbm.at[pl.ds(base, CH)], sidx.at[0])
        pltpu.sync_copy(dst_hbm.at[pl.ds(base, CH)], didx.at[0])
        pltpu.async_copy(hs_hbm.at[sidx.at[0]], rows, sem).wait()
        pltpu.sync_copy(rows, sh_acc.at[didx.at[0]], add=True)
        return 0

    lax.fori_loop(0, cnt, body, 0)
    plsc.subcore_barrier()
    pltpu.sync_copy(
        sh_acc.at[pl.ds(r0, ROWS_PER_TILE)],
        out_hbm.at[c, pl.ds(r0, ROWS_PER_TILE)],
    )


@functools.partial(
    pl.kernel,
    out_type=jax.ShapeDtypeStruct((E,), jnp.float32),
    mesh=_mesh,
    compiler_params=pltpu.CompilerParams(needs_layout_passes=False),
    scratch_types=[
        pltpu.VMEM((1, CH), jnp.int32),
        pltpu.VMEM((1, CH), jnp.int32),
        pltpu.VMEM((CH, H), jnp.float32),
        pltpu.VMEM((CH, H), jnp.float32),
        pltpu.VMEM((CH,), jnp.float32),
        pltpu.SemaphoreType.DMA,
        pltpu.SemaphoreType.DMA,
    ],
)
def _score_kernel(h2_hbm, src_hbm, dst_hbm, out_hbm,
                  sidx, didx, arows, brows, sbuf, sema, semb):
    c = lax.axis_index("c")
    s = lax.axis_index("s")
    wid = s * NC + c
    start, cnt = _worker_chunks(wid)
    lanes = lax.iota(jnp.int32, L)

    def chunk_body(t, _):
        base = (start + t) * CH
        pltpu.sync_copy(src_hbm.at[pl.ds(base, CH)], sidx.at[0])
        pltpu.sync_copy(dst_hbm.at[pl.ds(base, CH)], didx.at[0])
        cpa = pltpu.async_copy(h2_hbm.at[sidx.at[0]], arows, sema)
        cpb = pltpu.async_copy(h2_hbm.at[didx.at[0]], brows, semb)
        cpa.wait()
        cpb.wait()

        def group_body(g, _):
            eids = lanes + g * L
            acc = jnp.zeros((L,), dtype=jnp.float32)
            for f in range(H):
                fv = jnp.full((L,), f, dtype=jnp.int32)
                a = plsc.load_gather(arows, [eids, fv])
                b = plsc.load_gather(brows, [eids, fv])
                acc = acc + a * b
            sbuf[pl.ds(g * L, L)] = acc
            return 0

        lax.fori_loop(0, CH // L, group_body, 0)
        pltpu.sync_copy(sbuf, out_hbm.at[pl.ds(base, CH)])
        return 0

    lax.fori_loop(0, cnt, chunk_body, 0)


def _minmax_body(x_ref, mn_ref, mx_ref):
    i = pl.program_id(0)
    x = x_ref[...]
    bmn = jnp.min(x, axis=0, keepdims=True)
    bmx = jnp.max(x, axis=0, keepdims=True)

    @pl.when(i == 0)
    def _():
        mn_ref[...] = bmn
        mx_ref[...] = bmx

    @pl.when(i > 0)
    def _():
        mn_ref[...] = jnp.minimum(mn_ref[...], bmn)
        mx_ref[...] = jnp.maximum(mx_ref[...], bmx)


def _dense1_body(x_ref, mn_ref, mx_ref, degc_ref, w1_ref, hs1_ref, dinv_ref):
    mn = mn_ref[...]
    mx = mx_ref[...]
    den = jnp.where(mx > mn, mx - mn, jnp.ones_like(mx))
    xf = (x_ref[...] - mn) / den
    dinv = lax.rsqrt(degc_ref[...] + 1.0)
    dinv_ref[...] = dinv
    h = lax.dot_general(xf, w1_ref[...], (((1,), (1,)), ((), ())),
                        preferred_element_type=jnp.float32)
    hs1_ref[...] = h * dinv


def _dense2_body(acc_ref, hs1_ref, dinv_ref, b1_ref, w2_ref, hs2_ref):
    dinv = dinv_ref[...]
    agg = acc_ref[0] + acc_ref[1] + hs1_ref[...]
    h1 = jnp.maximum(agg * dinv + b1_ref[...], 0.0)
    h = lax.dot_general(h1, w2_ref[...], (((1,), (1,)), ((), ())),
                        preferred_element_type=jnp.float32)
    hs2_ref[...] = h * dinv


def _dense3_body(acc_ref, hs2_ref, dinv_ref, b2_ref, h2_ref):
    agg = acc_ref[0] + acc_ref[1] + hs2_ref[...]
    h2_ref[...] = agg * dinv_ref[...] + b2_ref[...]


_RB = 2000
_G = N // _RB


def kernel(x, edge_index, W1, b1, W2, b2):
    src = edge_index[0]
    dst = edge_index[1]
    zeros1 = jnp.zeros((DEG_STRIPE,), dtype=jnp.float32)
    zeros2 = jnp.zeros((ROWS_PER_TILE, H), dtype=jnp.float32)

    degp = _deg_kernel(dst, zeros1).reshape(NC, DEGP)
    degc = (degp[0, :N] + degp[1, :N]).reshape(N, 1)

    mn, mx = pl.pallas_call(
        _minmax_body,
        grid=(_G,),
        in_specs=[pl.BlockSpec((_RB, D), lambda i: (i, 0))],
        out_specs=[pl.BlockSpec((1, D), lambda i: (0, 0)),
                   pl.BlockSpec((1, D), lambda i: (0, 0))],
        out_shape=[jax.ShapeDtypeStruct((1, D), jnp.float32),
                   jax.ShapeDtypeStruct((1, D), jnp.float32)],
    )(x)

    hs1, dinv = pl.pallas_call(
        _dense1_body,
        grid=(_G,),
        in_specs=[
            pl.BlockSpec((_RB, D), lambda i: (i, 0)),
            pl.BlockSpec((1, D), lambda i: (0, 0)),
            pl.BlockSpec((1, D), lambda i: (0, 0)),
            pl.BlockSpec((_RB, 1), lambda i: (i, 0)),
            pl.BlockSpec((H, D), lambda i: (0, 0)),
        ],
        out_specs=[pl.BlockSpec((_RB, H), lambda i: (i, 0)),
                   pl.BlockSpec((_RB, 1), lambda i: (i, 0))],
        out_shape=[jax.ShapeDtypeStruct((N, H), jnp.float32),
                   jax.ShapeDtypeStruct((N, 1), jnp.float32)],
    )(x, mn, mx, degc, W1)

    acc1 = _agg_kernel(hs1, src, dst, zeros2)[:, :N]

    hs2 = pl.pallas_call(
        _dense2_body,
        grid=(_G,),
        in_specs=[
            pl.BlockSpec((NC, _RB, H), lambda i: (0, i, 0)),
            pl.BlockSpec((_RB, H), lambda i: (i, 0)),
            pl.BlockSpec((_RB, 1), lambda i: (i, 0)),
            pl.BlockSpec((H,), lambda i: (0,)),
            pl.BlockSpec((H, H), lambda i: (0, 0)),
        ],
        out_specs=pl.BlockSpec((_RB, H), lambda i: (i, 0)),
        out_shape=jax.ShapeDtypeStruct((N, H), jnp.float32),
    )(acc1, hs1, dinv, b1, W2)

    acc2 = _agg_kernel(hs2, src, dst, zeros2)[:, :N]

    h2 = pl.pallas_call(
        _dense3_body,
        grid=(_G,),
        in_specs=[
            pl.BlockSpec((NC, _RB, H), lambda i: (0, i, 0)),
            pl.BlockSpec((_RB, H), lambda i: (i, 0)),
            pl.BlockSpec((_RB, 1), lambda i: (i, 0)),
            pl.BlockSpec((H,), lambda i: (0,)),
        ],
        out_specs=pl.BlockSpec((_RB, H), lambda i: (i, 0)),
        out_shape=jax.ShapeDtypeStruct((N, H), jnp.float32),
    )(acc2, hs2, dinv, b2)

    return _score_kernel(h2, src, dst)


# speedup vs baseline: 12.8458x; 2.5857x over previous
import functools

import jax
import jax.numpy as jnp
from jax import lax
from jax.experimental import pallas as pl
from jax.experimental.pallas import tpu as pltpu
from jax.experimental.pallas import tpu_sc as plsc

N = 10000
E = 320000
D = 128
H = 128

NC = 2
NS = 16
NW = NC * NS
L = 16

CH = 128
NCHUNK = E // CH
CH_BASE = NCHUNK // NW
CH_REM = NCHUNK % NW

NPAD = 10112
ROWS_PER_TILE = NPAD // NS
DEG_STRIPE = 640
DEGP = DEG_STRIPE * NS

_mesh = plsc.VectorSubcoreMesh(
    core_axis_name="c", subcore_axis_name="s", num_cores=NC, num_subcores=NS
)


def _worker_chunks(wid):
    cnt = CH_BASE + jnp.where(wid < CH_REM, 1, 0)
    start = wid * CH_BASE + jnp.minimum(wid, CH_REM)
    return start, cnt


@functools.partial(
    pl.kernel,
    out_type=jax.ShapeDtypeStruct((NC * DEGP,), jnp.float32),
    mesh=_mesh,
    compiler_params=pltpu.CompilerParams(needs_layout_passes=False),
    scratch_types=[
        pltpu.VMEM((1, CH), jnp.int32),
        pltpu.VMEM((CH,), jnp.float32),
        pltpu.VMEM_SHARED((DEGP,), jnp.float32),
    ],
)
def _deg_kernel(dst_hbm, zeros1_hbm, out_hbm, didx, ones_v, sh_deg):
    c = lax.axis_index("c")
    s = lax.axis_index("s")
    wid = s * NC + c
    one = jnp.full((L,), 1.0, dtype=jnp.float32)
    for k in range(CH // L):
        ones_v[pl.ds(k * L, L)] = one
    pltpu.sync_copy(
        zeros1_hbm.at[pl.ds(0, DEG_STRIPE)],
        sh_deg.at[pl.ds(s * DEG_STRIPE, DEG_STRIPE)],
    )
    plsc.subcore_barrier()

    start, cnt = _worker_chunks(wid)

    def body(t, _):
        base = (start + t) * CH
        pltpu.sync_copy(dst_hbm.at[pl.ds(base, CH)], didx.at[0])
        pltpu.sync_copy(ones_v, sh_deg.at[didx.at[0]], add=True)
        return 0

    lax.fori_loop(0, cnt, body, 0)
    plsc.subcore_barrier()
    pltpu.sync_copy(
        sh_deg.at[pl.ds(s * DEG_STRIPE, DEG_STRIPE)],
        out_hbm.at[pl.ds(c * DEGP + s * DEG_STRIPE, DEG_STRIPE)],
    )


@functools.partial(
    pl.kernel,
    out_type=jax.ShapeDtypeStruct((NC, NPAD, H), jnp.float32),
    mesh=_mesh,
    compiler_params=pltpu.CompilerParams(needs_layout_passes=False),
    scratch_types=[
        pltpu.VMEM((2, CH), jnp.int32),
        pltpu.VMEM((2, CH), jnp.int32),
        pltpu.VMEM((CH, H), jnp.float32),
        pltpu.VMEM((CH, H), jnp.float32),
        pltpu.VMEM_SHARED((NPAD, H), jnp.float32),
        pltpu.SemaphoreType.DMA,
        pltpu.SemaphoreType.DMA,
    ],
)
def _agg_kernel(hs_hbm, src_hbm, dst_hbm, zeros2_hbm, out_hbm,
                sidx, didx, rows0, rows1, sh_acc, sem0, sem1):
    c = lax.axis_index("c")
    s = lax.axis_index("s")
    wid = s * NC + c
    r0 = s * ROWS_PER_TILE
    pltpu.sync_copy(
        zeros2_hbm.at[pl.ds(0, ROWS_PER_TILE)],
        sh_acc.at[pl.ds(r0, ROWS_PER_TILE)],
    )
    plsc.subcore_barrier()

    start, cnt = _worker_chunks(wid)
    rows = (rows0, rows1)
    sems = (sem0, sem1)

    def fetch(t, b):
        base = (start + t) * CH
        pltpu.sync_copy(src_hbm.at[pl.ds(base, CH)], sidx.at[b])
        pltpu.sync_copy(dst_hbm.at[pl.ds(base, CH)], didx.at[b])
        pltpu.async_copy(hs_hbm.at[sidx.at[b]], rows[b], sems[b])

    @pl.when(cnt > 0)
    def _():
        fetch(0, 0)

    def pair_body(p, _):
        for b in range(2):
            t = 2 * p + b
            nb = 1 - b

            @pl.when(t + 1 < cnt)
            def _():
                fetch(t + 1, nb)

            @pl.when(t < cnt)
            def _():
                pltpu.make_async_copy(
                    hs_hbm.at[sidx.at[b]], rows[b], sems[b]
                ).wait()
                pltpu.sync_copy(rows[b], sh_acc.at[didx.at[b]], add=True)
        return 0

    lax.fori_loop(0, (CH_BASE + 2) // 2, pair_body, 0)
    plsc.subcore_barrier()
    pltpu.sync_copy(
        sh_acc.at[pl.ds(r0, ROWS_PER_TILE)],
        out_hbm.at[c, pl.ds(r0, ROWS_PER_TILE)],
    )


@functools.partial(
    pl.kernel,
    out_type=jax.ShapeDtypeStruct((E,), jnp.float32),
    mesh=_mesh,
    compiler_params=pltpu.CompilerParams(needs_layout_passes=False),
    scratch_types=[
        pltpu.VMEM((2, CH), jnp.int32),
        pltpu.VMEM((2, CH), jnp.int32),
        pltpu.VMEM((CH, H), jnp.float32),
        pltpu.VMEM((CH, H), jnp.float32),
        pltpu.VMEM((CH, H), jnp.float32),
        pltpu.VMEM((CH, H), jnp.float32),
        pltpu.VMEM((CH,), jnp.float32),
        pltpu.SemaphoreType.DMA,
        pltpu.SemaphoreType.DMA,
        pltpu.SemaphoreType.DMA,
        pltpu.SemaphoreType.DMA,
    ],
)
def _score_kernel(h2_hbm, src_hbm, dst_hbm, out_hbm,
                  sidx, didx, arows0, arows1, brows0, brows1, sbuf,
                  sema0, sema1, semb0, semb1):
    c = lax.axis_index("c")
    s = lax.axis_index("s")
    wid = s * NC + c
    start, cnt = _worker_chunks(wid)
    lanes = lax.iota(jnp.int32, L)
    rots = [(lanes + r) % L for r in range(L)]
    arows = (arows0, arows1)
    brows = (brows0, brows1)
    semas = (sema0, sema1)
    sembs = (semb0, semb1)

    def fetch(t, b):
        base = (start + t) * CH
        pltpu.sync_copy(src_hbm.at[pl.ds(base, CH)], sidx.at[b])
        pltpu.sync_copy(dst_hbm.at[pl.ds(base, CH)], didx.at[b])
        pltpu.async_copy(h2_hbm.at[sidx.at[b]], arows[b], semas[b])
        pltpu.async_copy(h2_hbm.at[didx.at[b]], brows[b], sembs[b])

    @pl.when(cnt > 0)
    def _():
        fetch(0, 0)

    def pair_body(p, _):
        for b in range(2):
            t = 2 * p + b
            nb = 1 - b

            @pl.when(t + 1 < cnt)
            def _():
                fetch(t + 1, nb)

            @pl.when(t < cnt)
            def _():
                pltpu.make_async_copy(
                    h2_hbm.at[sidx.at[b]], arows[b], semas[b]
                ).wait()
                pltpu.make_async_copy(
                    h2_hbm.at[didx.at[b]], brows[b], sembs[b]
                ).wait()

                def group_body(g, _):
                    eids = lanes + g * L
                    acc = jnp.zeros((L,), dtype=jnp.float32)
                    for f in range(H):
                        colv = rots[f % L] + (f - f % L)
                        a = plsc.load_gather(arows[b], [eids, colv])
                        bb = plsc.load_gather(brows[b], [eids, colv])
                        acc = acc + a * bb
                    sbuf[pl.ds(g * L, L)] = acc
                    return 0

                lax.fori_loop(0, CH // L, group_body, 0)
                base = (start + t) * CH
                pltpu.sync_copy(sbuf, out_hbm.at[pl.ds(base, CH)])
        return 0

    lax.fori_loop(0, (CH_BASE + 2) // 2, pair_body, 0)


def _minmax_body(x_ref, mn_ref, mx_ref):
    i = pl.program_id(0)
    x = x_ref[...]
    bmn = jnp.min(x, axis=0, keepdims=True)
    bmx = jnp.max(x, axis=0, keepdims=True)

    @pl.when(i == 0)
    def _():
        mn_ref[...] = bmn
        mx_ref[...] = bmx

    @pl.when(i > 0)
    def _():
        mn_ref[...] = jnp.minimum(mn_ref[...], bmn)
        mx_ref[...] = jnp.maximum(mx_ref[...], bmx)


def _dense1_body(x_ref, mn_ref, mx_ref, degc_ref, w1_ref, hs1_ref, dinv_ref):
    mn = mn_ref[...]
    mx = mx_ref[...]
    den = jnp.where(mx > mn, mx - mn, jnp.ones_like(mx))
    xf = (x_ref[...] - mn) / den
    dinv = lax.rsqrt(degc_ref[...] + 1.0)
    dinv_ref[...] = dinv
    h = lax.dot_general(xf, w1_ref[...], (((1,), (1,)), ((), ())),
                        preferred_element_type=jnp.float32)
    hs1_ref[...] = h * dinv


def _dense2_body(acc_ref, hs1_ref, dinv_ref, b1_ref, w2_ref, hs2_ref):
    dinv = dinv_ref[...]
    agg = acc_ref[0] + acc_ref[1] + hs1_ref[...]
    h1 = jnp.maximum(agg * dinv + b1_ref[...], 0.0)
    h = lax.dot_general(h1, w2_ref[...], (((1,), (1,)), ((), ())),
                        preferred_element_type=jnp.float32)
    hs2_ref[...] = h * dinv


def _dense3_body(acc_ref, hs2_ref, dinv_ref, b2_ref, h2_ref):
    agg = acc_ref[0] + acc_ref[1] + hs2_ref[...]
    h2_ref[...] = agg * dinv_ref[...] + b2_ref[...]


_RB = 2000
_G = N // _RB


def kernel(x, edge_index, W1, b1, W2, b2):
    src = edge_index[0]
    dst = edge_index[1]
    zeros1 = jnp.zeros((DEG_STRIPE,), dtype=jnp.float32)
    zeros2 = jnp.zeros((ROWS_PER_TILE, H), dtype=jnp.float32)

    degp = _deg_kernel(dst, zeros1).reshape(NC, DEGP)
    degc = (degp[0, :N] + degp[1, :N]).reshape(N, 1)

    mn, mx = pl.pallas_call(
        _minmax_body,
        grid=(_G,),
        in_specs=[pl.BlockSpec((_RB, D), lambda i: (i, 0))],
        out_specs=[pl.BlockSpec((1, D), lambda i: (0, 0)),
                   pl.BlockSpec((1, D), lambda i: (0, 0))],
        out_shape=[jax.ShapeDtypeStruct((1, D), jnp.float32),
                   jax.ShapeDtypeStruct((1, D), jnp.float32)],
    )(x)

    hs1, dinv = pl.pallas_call(
        _dense1_body,
        grid=(_G,),
        in_specs=[
            pl.BlockSpec((_RB, D), lambda i: (i, 0)),
            pl.BlockSpec((1, D), lambda i: (0, 0)),
            pl.BlockSpec((1, D), lambda i: (0, 0)),
            pl.BlockSpec((_RB, 1), lambda i: (i, 0)),
            pl.BlockSpec((H, D), lambda i: (0, 0)),
        ],
        out_specs=[pl.BlockSpec((_RB, H), lambda i: (i, 0)),
                   pl.BlockSpec((_RB, 1), lambda i: (i, 0))],
        out_shape=[jax.ShapeDtypeStruct((N, H), jnp.float32),
                   jax.ShapeDtypeStruct((N, 1), jnp.float32)],
    )(x, mn, mx, degc, W1)

    acc1 = _agg_kernel(hs1, src, dst, zeros2)[:, :N]

    hs2 = pl.pallas_call(
        _dense2_body,
        grid=(_G,),
        in_specs=[
            pl.BlockSpec((NC, _RB, H), lambda i: (0, i, 0)),
            pl.BlockSpec((_RB, H), lambda i: (i, 0)),
            pl.BlockSpec((_RB, 1), lambda i: (i, 0)),
            pl.BlockSpec((H,), lambda i: (0,)),
            pl.BlockSpec((H, H), lambda i: (0, 0)),
        ],
        out_specs=pl.BlockSpec((_RB, H), lambda i: (i, 0)),
        out_shape=jax.ShapeDtypeStruct((N, H), jnp.float32),
    )(acc1, hs1, dinv, b1, W2)

    acc2 = _agg_kernel(hs2, src, dst, zeros2)[:, :N]

    h2 = pl.pallas_call(
        _dense3_body,
        grid=(_G,),
        in_specs=[
            pl.BlockSpec((NC, _RB, H), lambda i: (0, i, 0)),
            pl.BlockSpec((_RB, H), lambda i: (i, 0)),
            pl.BlockSpec((_RB, 1), lambda i: (i, 0)),
            pl.BlockSpec((H,), lambda i: (0,)),
        ],
        out_specs=pl.BlockSpec((_RB, H), lambda i: (i, 0)),
        out_shape=jax.ShapeDtypeStruct((N, H), jnp.float32),
    )(acc2, hs2, dinv, b2)

    return _score_kernel(h2, src, dst)


# speedup vs baseline: 20.1199x; 1.5663x over previous
import functools

import jax
import jax.numpy as jnp
from jax import lax
from jax.experimental import pallas as pl
from jax.experimental.pallas import tpu as pltpu
from jax.experimental.pallas import tpu_sc as plsc

N = 10000
E = 320000
D = 128
H = 128

NC = 2
NS = 16
NW = NC * NS
L = 16

CH = 128
BLK = 8
NCHUNK_PAD = 2504
E_PAD = NCHUNK_PAD * CH
NBLK = NCHUNK_PAD // BLK
BLK_BASE = NBLK // NW
BLK_REM = NBLK % NW

NPAD = 10112
ROWS_PER_TILE = NPAD // NS
DEG_STRIPE = 640
DEGP = DEG_STRIPE * NS

_mesh = plsc.VectorSubcoreMesh(
    core_axis_name="c", subcore_axis_name="s", num_cores=NC, num_subcores=NS
)
_cparams = pltpu.CompilerParams(needs_layout_passes=False)


def _worker_blocks(wid):
    cnt = BLK_BASE + jnp.where(wid < BLK_REM, 1, 0)
    start = wid * BLK_BASE + jnp.minimum(wid, BLK_REM)
    return start, cnt


@functools.partial(
    pl.kernel,
    out_type=jax.ShapeDtypeStruct((NC * DEGP,), jnp.float32),
    mesh=_mesh,
    compiler_params=_cparams,
    scratch_types=[
        pltpu.VMEM((2, BLK, CH), jnp.int32),
        pltpu.VMEM((CH,), jnp.float32),
        pltpu.VMEM_SHARED((DEGP,), jnp.float32),
        pltpu.SemaphoreType.DMA,
    ],
)
def _deg_kernel(dst2d_hbm, zeros1_hbm, out_hbm, didxb, ones_v, sh_deg, sem):
    c = lax.axis_index("c")
    s = lax.axis_index("s")
    wid = s * NC + c
    one = jnp.full((L,), 1.0, dtype=jnp.float32)
    for k in range(CH // L):
        ones_v[pl.ds(k * L, L)] = one
    pltpu.sync_copy(
        zeros1_hbm.at[pl.ds(0, DEG_STRIPE)],
        sh_deg.at[pl.ds(s * DEG_STRIPE, DEG_STRIPE)],
    )
    plsc.subcore_barrier()

    start, cnt = _worker_blocks(wid)
    pltpu.sync_copy(dst2d_hbm.at[pl.ds(start * BLK, BLK)], didxb.at[0])

    def blk_body(p, _):
        ib = p % 2

        @pl.when(p > 0)
        def _():
            for r in range(BLK):
                pltpu.make_async_copy(
                    ones_v, sh_deg.at[didxb.at[1 - ib, r]], sem
                ).wait()

        @pl.when(p + 1 < cnt)
        def _():
            pltpu.sync_copy(
                dst2d_hbm.at[pl.ds((start + p + 1) * BLK, BLK)],
                didxb.at[1 - ib],
            )

        for r in range(BLK):
            pltpu.async_copy(ones_v, sh_deg.at[didxb.at[ib, r]], sem,
                             add=True)
        return 0

    lax.fori_loop(0, cnt, blk_body, 0)
    for r in range(BLK):
        pltpu.make_async_copy(ones_v, sh_deg.at[didxb.at[0, r]], sem).wait()
    plsc.subcore_barrier()
    pltpu.sync_copy(
        sh_deg.at[pl.ds(s * DEG_STRIPE, DEG_STRIPE)],
        out_hbm.at[pl.ds(c * DEGP + s * DEG_STRIPE, DEG_STRIPE)],
    )


@functools.partial(
    pl.kernel,
    out_type=jax.ShapeDtypeStruct((NC, NPAD, H), jnp.float32),
    mesh=_mesh,
    compiler_params=_cparams,
    scratch_types=[
        pltpu.VMEM((2, BLK, CH), jnp.int32),
        pltpu.VMEM((2, BLK, CH), jnp.int32),
        pltpu.VMEM((CH, H), jnp.float32),
        pltpu.VMEM((CH, H), jnp.float32),
        pltpu.VMEM_SHARED((NPAD, H), jnp.float32),
        pltpu.SemaphoreType.DMA,
        pltpu.SemaphoreType.DMA,
        pltpu.SemaphoreType.DMA,
        pltpu.SemaphoreType.DMA,
    ],
)
def _agg_kernel(hs_hbm, src2d_hbm, dst2d_hbm, zeros2_hbm, out_hbm,
                sidxb, didxb, rows0, rows1, sh_acc,
                semg0, semg1, sems0, sems1):
    c = lax.axis_index("c")
    s = lax.axis_index("s")
    wid = s * NC + c
    r0 = s * ROWS_PER_TILE
    pltpu.sync_copy(
        zeros2_hbm.at[pl.ds(0, ROWS_PER_TILE)],
        sh_acc.at[pl.ds(r0, ROWS_PER_TILE)],
    )
    plsc.subcore_barrier()

    start, cnt = _worker_blocks(wid)
    rows = (rows0, rows1)
    semg = (semg0, semg1)
    sems = (sems0, sems1)

    pltpu.sync_copy(src2d_hbm.at[pl.ds(start * BLK, BLK)], sidxb.at[0])
    pltpu.sync_copy(dst2d_hbm.at[pl.ds(start * BLK, BLK)], didxb.at[0])
    pltpu.async_copy(hs_hbm.at[sidxb.at[0, 0]], rows[0], semg[0])

    def blk_body(p, _):
        ib = p % 2

        @pl.when(p > 0)
        def _():
            pltpu.make_async_copy(
                rows[1], sh_acc.at[didxb.at[ib, 0]], sems[1]
            ).wait()

        @pl.when(p + 1 < cnt)
        def _():
            pltpu.sync_copy(
                src2d_hbm.at[pl.ds((start + p + 1) * BLK, BLK)],
                sidxb.at[1 - ib],
            )
            pltpu.sync_copy(
                dst2d_hbm.at[pl.ds((start + p + 1) * BLK, BLK)],
                didxb.at[1 - ib],
            )

        for r in range(BLK):
            b = r % 2
            nb = 1 - b
            if r < BLK - 1:
                if r >= 1:
                    pltpu.make_async_copy(
                        rows[nb], sh_acc.at[didxb.at[ib, 0]], sems[nb]
                    ).wait()
                pltpu.async_copy(
                    hs_hbm.at[sidxb.at[ib, r + 1]], rows[nb], semg[nb]
                )
            else:
                @pl.when(p + 1 < cnt)
                def _():
                    pltpu.make_async_copy(
                        rows[0], sh_acc.at[didxb.at[ib, 0]], sems[0]
                    ).wait()
                    pltpu.async_copy(
                        hs_hbm.at[sidxb.at[1 - ib, 0]], rows[0], semg[0]
                    )
            pltpu.make_async_copy(
                hs_hbm.at[sidxb.at[ib, r]], rows[b], semg[b]
            ).wait()
            pltpu.async_copy(
                rows[b], sh_acc.at[didxb.at[ib, r]], sems[b], add=True
            )
        return 0

    lax.fori_loop(0, cnt, blk_body, 0)
    for b in range(2):
        pltpu.make_async_copy(
            rows[b], sh_acc.at[didxb.at[0, 0]], sems[b]
        ).wait()
    plsc.subcore_barrier()
    pltpu.sync_copy(
        sh_acc.at[pl.ds(r0, ROWS_PER_TILE)],
        out_hbm.at[c, pl.ds(r0, ROWS_PER_TILE)],
    )


@functools.partial(
    pl.kernel,
    out_type=jax.ShapeDtypeStruct((E_PAD,), jnp.float32),
    mesh=_mesh,
    compiler_params=_cparams,
    scratch_types=[
        pltpu.VMEM((2, BLK, CH), jnp.int32),
        pltpu.VMEM((2, BLK, CH), jnp.int32),
        pltpu.VMEM((CH, H), jnp.float32),
        pltpu.VMEM((CH, H), jnp.float32),
        pltpu.VMEM((CH, H), jnp.float32),
        pltpu.VMEM((CH, H), jnp.float32),
        pltpu.VMEM((CH,), jnp.float32),
        pltpu.SemaphoreType.DMA,
        pltpu.SemaphoreType.DMA,
        pltpu.SemaphoreType.DMA,
        pltpu.SemaphoreType.DMA,
    ],
)
def _score_kernel(h2_hbm, src2d_hbm, dst2d_hbm, out_hbm,
                  sidxb, didxb, arows0, arows1, brows0, brows1, sbuf,
                  sema0, sema1, semb0, semb1):
    c = lax.axis_index("c")
    s = lax.axis_index("s")
    wid = s * NC + c
    start, cnt = _worker_blocks(wid)
    lanes = lax.iota(jnp.int32, L)
    rots = [(lanes + r) % L for r in range(L)]
    arows = (arows0, arows1)
    brows = (brows0, brows1)
    sema = (sema0, sema1)
    semb = (semb0, semb1)

    def fire(ibx, rx, b):
        pltpu.async_copy(h2_hbm.at[sidxb.at[ibx, rx]], arows[b], sema[b])
        pltpu.async_copy(h2_hbm.at[didxb.at[ibx, rx]], brows[b], semb[b])

    def drain(b):
        pltpu.make_async_copy(h2_hbm.at[sidxb.at[0, 0]], arows[b],
                              sema[b]).wait()
        pltpu.make_async_copy(h2_hbm.at[didxb.at[0, 0]], brows[b],
                              semb[b]).wait()

    pltpu.sync_copy(src2d_hbm.at[pl.ds(start * BLK, BLK)], sidxb.at[0])
    pltpu.sync_copy(dst2d_hbm.at[pl.ds(start * BLK, BLK)], didxb.at[0])
    fire(0, 0, 0)

    def compute_chunk(b, out_base):
        def group_body(g, _):
            eids = lanes + g * L

            def fblk_body(k, acc):
                fbase = k * L
                for j in range(L):
                    colv = rots[j] + fbase
                    a = plsc.load_gather(arows[b], [eids, colv])
                    bb = plsc.load_gather(brows[b], [eids, colv])
                    acc = acc + a * bb
                return acc

            acc = lax.fori_loop(
                0, H // L, fblk_body, jnp.zeros((L,), dtype=jnp.float32))
            sbuf[pl.ds(g * L, L)] = acc
            return 0

        lax.fori_loop(0, CH // L, group_body, 0)
        pltpu.sync_copy(sbuf, out_hbm.at[pl.ds(out_base, CH)])

    def blk_body(p, _):
        ib = p % 2

        @pl.when(p + 1 < cnt)
        def _():
            pltpu.sync_copy(
                src2d_hbm.at[pl.ds((start + p + 1) * BLK, BLK)],
                sidxb.at[1 - ib],
            )
            pltpu.sync_copy(
                dst2d_hbm.at[pl.ds((start + p + 1) * BLK, BLK)],
                didxb.at[1 - ib],
            )

        for r in range(BLK):
            b = r % 2
            nb = 1 - b
            if r < BLK - 1:
                fire(ib, r + 1, nb)
            else:
                @pl.when(p + 1 < cnt)
                def _():
                    fire(1 - ib, 0, nb)
            drain(b)
            compute_chunk(b, ((start + p) * BLK + r) * CH)
        return 0

    lax.fori_loop(0, cnt, blk_body, 0)


def _minmax_body(x_ref, mn_ref, mx_ref):
    i = pl.program_id(0)
    x = x_ref[...]
    bmn = jnp.min(x, axis=0, keepdims=True)
    bmx = jnp.max(x, axis=0, keepdims=True)

    @pl.when(i == 0)
    def _():
        mn_ref[...] = bmn
        mx_ref[...] = bmx

    @pl.when(i > 0)
    def _():
        mn_ref[...] = jnp.minimum(mn_ref[...], bmn)
        mx_ref[...] = jnp.maximum(mx_ref[...], bmx)


def _dense1_body(x_ref, mn_ref, mx_ref, degc_ref, w1_ref, hs1_ref, dinv_ref):
    mn = mn_ref[...]
    mx = mx_ref[...]
    den = jnp.where(mx > mn, mx - mn, jnp.ones_like(mx))
    xf = (x_ref[...] - mn) / den
    dinv = lax.rsqrt(degc_ref[...] + 1.0)
    dinv_ref[...] = dinv
    h = lax.dot_general(xf, w1_ref[...], (((1,), (1,)), ((), ())),
                        preferred_element_type=jnp.float32)
    hs1_ref[...] = h * dinv


def _dense2_body(acc_ref, hs1_ref, dinv_ref, b1_ref, w2_ref, hs2_ref):
    dinv = dinv_ref[...]
    agg = acc_ref[0] + acc_ref[1] + hs1_ref[...]
    h1 = jnp.maximum(agg * dinv + b1_ref[...], 0.0)
    h = lax.dot_general(h1, w2_ref[...], (((1,), (1,)), ((), ())),
                        preferred_element_type=jnp.float32)
    hs2_ref[...] = h * dinv


def _dense3_body(acc_ref, hs2_ref, dinv_ref, b2_ref, h2_ref):
    agg = acc_ref[0] + acc_ref[1] + hs2_ref[...]
    h2_ref[...] = agg * dinv_ref[...] + b2_ref[...]


_RB = 2000
_G = N // _RB


def kernel(x, edge_index, W1, b1, W2, b2):
    src = edge_index[0]
    dst = edge_index[1]
    srcp = jnp.pad(src, (0, E_PAD - E)).reshape(NCHUNK_PAD, CH)
    dstp = jnp.pad(dst, (0, E_PAD - E), constant_values=N).reshape(
        NCHUNK_PAD, CH)
    zeros1 = jnp.zeros((DEG_STRIPE,), dtype=jnp.float32)
    zeros2 = jnp.zeros((ROWS_PER_TILE, H), dtype=jnp.float32)

    degp = _deg_kernel(dstp, zeros1).reshape(NC, DEGP)
    degc = (degp[0, :N] + degp[1, :N]).reshape(N, 1)

    mn, mx = pl.pallas_call(
        _minmax_body,
        grid=(_G,),
        in_specs=[pl.BlockSpec((_RB, D), lambda i: (i, 0))],
        out_specs=[pl.BlockSpec((1, D), lambda i: (0, 0)),
                   pl.BlockSpec((1, D), lambda i: (0, 0))],
        out_shape=[jax.ShapeDtypeStruct((1, D), jnp.float32),
                   jax.ShapeDtypeStruct((1, D), jnp.float32)],
    )(x)

    hs1, dinv = pl.pallas_call(
        _dense1_body,
        grid=(_G,),
        in_specs=[
            pl.BlockSpec((_RB, D), lambda i: (i, 0)),
            pl.BlockSpec((1, D), lambda i: (0, 0)),
            pl.BlockSpec((1, D), lambda i: (0, 0)),
            pl.BlockSpec((_RB, 1), lambda i: (i, 0)),
            pl.BlockSpec((H, D), lambda i: (0, 0)),
        ],
        out_specs=[pl.BlockSpec((_RB, H), lambda i: (i, 0)),
                   pl.BlockSpec((_RB, 1), lambda i: (i, 0))],
        out_shape=[jax.ShapeDtypeStruct((N, H), jnp.float32),
                   jax.ShapeDtypeStruct((N, 1), jnp.float32)],
    )(x, mn, mx, degc, W1)

    acc1 = _agg_kernel(hs1, srcp, dstp, zeros2)[:, :N]

    hs2 = pl.pallas_call(
        _dense2_body,
        grid=(_G,),
        in_specs=[
            pl.BlockSpec((NC, _RB, H), lambda i: (0, i, 0)),
            pl.BlockSpec((_RB, H), lambda i: (i, 0)),
            pl.BlockSpec((_RB, 1), lambda i: (i, 0)),
            pl.BlockSpec((H,), lambda i: (0,)),
            pl.BlockSpec((H, H), lambda i: (0, 0)),
        ],
        out_specs=pl.BlockSpec((_RB, H), lambda i: (i, 0)),
        out_shape=jax.ShapeDtypeStruct((N, H), jnp.float32),
    )(acc1, hs1, dinv, b1, W2)

    acc2 = _agg_kernel(hs2, srcp, dstp, zeros2)[:, :N]

    h2 = pl.pallas_call(
        _dense3_body,
        grid=(_G,),
        in_specs=[
            pl.BlockSpec((NC, _RB, H), lambda i: (0, i, 0)),
            pl.BlockSpec((_RB, H), lambda i: (i, 0)),
            pl.BlockSpec((_RB, 1), lambda i: (i, 0)),
            pl.BlockSpec((H,), lambda i: (0,)),
        ],
        out_specs=pl.BlockSpec((_RB, H), lambda i: (i, 0)),
        out_shape=jax.ShapeDtypeStruct((N, H), jnp.float32),
    )(acc2, hs2, dinv, b2)

    return _score_kernel(h2, srcp, dstp)[:E]


# speedup vs baseline: 20.4339x; 1.0156x over previous
import functools

import jax
import jax.numpy as jnp
from jax import lax
from jax.experimental import pallas as pl
from jax.experimental.pallas import tpu as pltpu
from jax.experimental.pallas import tpu_sc as plsc

N = 10000
E = 320000
D = 128
H = 128

NC = 2
NS = 16
NW = NC * NS
L = 16

CH = 128
BLK = 8
NCHUNK_PAD = 2504
E_PAD = NCHUNK_PAD * CH
NBLK = NCHUNK_PAD // BLK
BLK_BASE = NBLK // NW
BLK_REM = NBLK % NW

NPAD = 10112
ROWS_PER_TILE = NPAD // NS
DEG_STRIPE = 640
DEGP = DEG_STRIPE * NS

_mesh = plsc.VectorSubcoreMesh(
    core_axis_name="c", subcore_axis_name="s", num_cores=NC, num_subcores=NS
)
_cparams = pltpu.CompilerParams(needs_layout_passes=False)


def _worker_blocks(wid):
    cnt = BLK_BASE + jnp.where(wid < BLK_REM, 1, 0)
    start = wid * BLK_BASE + jnp.minimum(wid, BLK_REM)
    return start, cnt


@functools.partial(
    pl.kernel,
    out_type=jax.ShapeDtypeStruct((NC * DEGP,), jnp.float32),
    mesh=_mesh,
    compiler_params=_cparams,
    scratch_types=[
        pltpu.VMEM((2, BLK, CH), jnp.int32),
        pltpu.VMEM((CH,), jnp.float32),
        pltpu.VMEM_SHARED((DEGP,), jnp.float32),
        pltpu.SemaphoreType.DMA,
    ],
)
def _deg_kernel(dst2d_hbm, zeros1_hbm, out_hbm, didxb, ones_v, sh_deg, sem):
    c = lax.axis_index("c")
    s = lax.axis_index("s")
    wid = s * NC + c
    one = jnp.full((L,), 1.0, dtype=jnp.float32)
    for k in range(CH // L):
        ones_v[pl.ds(k * L, L)] = one
    pltpu.sync_copy(
        zeros1_hbm.at[pl.ds(0, DEG_STRIPE)],
        sh_deg.at[pl.ds(s * DEG_STRIPE, DEG_STRIPE)],
    )
    plsc.subcore_barrier()

    start, cnt = _worker_blocks(wid)
    pltpu.sync_copy(dst2d_hbm.at[pl.ds(start * BLK, BLK)], didxb.at[0])

    def blk_body(p, _):
        ib = p % 2

        @pl.when(p > 0)
        def _():
            for r in range(BLK):
                pltpu.make_async_copy(
                    ones_v, sh_deg.at[didxb.at[1 - ib, r]], sem
                ).wait()

        @pl.when(p + 1 < cnt)
        def _():
            pltpu.sync_copy(
                dst2d_hbm.at[pl.ds((start + p + 1) * BLK, BLK)],
                didxb.at[1 - ib],
            )

        for r in range(BLK):
            pltpu.async_copy(ones_v, sh_deg.at[didxb.at[ib, r]], sem,
                             add=True)
        return 0

    lax.fori_loop(0, cnt, blk_body, 0)
    for r in range(BLK):
        pltpu.make_async_copy(ones_v, sh_deg.at[didxb.at[0, r]], sem).wait()
    plsc.subcore_barrier()
    pltpu.sync_copy(
        sh_deg.at[pl.ds(s * DEG_STRIPE, DEG_STRIPE)],
        out_hbm.at[pl.ds(c * DEGP + s * DEG_STRIPE, DEG_STRIPE)],
    )


@functools.partial(
    pl.kernel,
    out_type=jax.ShapeDtypeStruct((NC, NPAD, H), jnp.float32),
    mesh=_mesh,
    compiler_params=_cparams,
    scratch_types=[
        pltpu.VMEM((2, BLK, CH), jnp.int32),
        pltpu.VMEM((2, BLK, CH), jnp.int32),
        pltpu.VMEM((CH, H), jnp.float32),
        pltpu.VMEM((CH, H), jnp.float32),
        pltpu.VMEM_SHARED((NPAD, H), jnp.float32),
        pltpu.SemaphoreType.DMA,
        pltpu.SemaphoreType.DMA,
        pltpu.SemaphoreType.DMA,
        pltpu.SemaphoreType.DMA,
    ],
)
def _agg_kernel(hs_hbm, src2d_hbm, dst2d_hbm, zeros2_hbm, out_hbm,
                sidxb, didxb, rows0, rows1, sh_acc,
                semg0, semg1, sems0, sems1):
    c = lax.axis_index("c")
    s = lax.axis_index("s")
    wid = s * NC + c
    r0 = s * ROWS_PER_TILE
    pltpu.sync_copy(
        zeros2_hbm.at[pl.ds(0, ROWS_PER_TILE)],
        sh_acc.at[pl.ds(r0, ROWS_PER_TILE)],
    )
    plsc.subcore_barrier()

    start, cnt = _worker_blocks(wid)
    rows = (rows0, rows1)
    semg = (semg0, semg1)
    sems = (sems0, sems1)

    def drain_scatter(b):
        pltpu.make_async_copy(
            rows[b], sh_acc.at[didxb.at[0, 0]], sems[b]
        ).wait()

    pltpu.sync_copy(src2d_hbm.at[pl.ds(start * BLK, BLK)], sidxb.at[0])
    pltpu.sync_copy(dst2d_hbm.at[pl.ds(start * BLK, BLK)], didxb.at[0])
    pltpu.async_copy(hs_hbm.at[sidxb.at[0, 0]], rows[0], semg[0])

    def blk_body(p, _):
        ib = p % 2

        @pl.when(p > 0)
        def _():
            drain_scatter(1)

        @pl.when(p + 1 < cnt)
        def _():
            pltpu.sync_copy(
                src2d_hbm.at[pl.ds((start + p + 1) * BLK, BLK)],
                sidxb.at[1 - ib],
            )
            pltpu.sync_copy(
                dst2d_hbm.at[pl.ds((start + p + 1) * BLK, BLK)],
                didxb.at[1 - ib],
            )

        for r in range(BLK):
            b = r % 2
            nb = 1 - b
            if r < BLK - 1:
                if r >= 1:
                    drain_scatter(nb)
                pltpu.async_copy(
                    hs_hbm.at[sidxb.at[ib, r + 1]], rows[nb], semg[nb]
                )
            else:
                @pl.when(p + 1 < cnt)
                def _():
                    drain_scatter(0)
                    pltpu.async_copy(
                        hs_hbm.at[sidxb.at[1 - ib, 0]], rows[0], semg[0]
                    )
            pltpu.make_async_copy(
                hs_hbm.at[sidxb.at[ib, r]], rows[b], semg[b]
            ).wait()
            pltpu.async_copy(
                rows[b], sh_acc.at[didxb.at[ib, r]], sems[b], add=True
            )
        return 0

    lax.fori_loop(0, cnt, blk_body, 0)
    for b in range(2):
        drain_scatter(b)
    plsc.subcore_barrier()
    pltpu.sync_copy(
        sh_acc.at[pl.ds(r0, ROWS_PER_TILE)],
        out_hbm.at[c, pl.ds(r0, ROWS_PER_TILE)],
    )


@functools.partial(
    pl.kernel,
    out_type=jax.ShapeDtypeStruct((E_PAD,), jnp.float32),
    mesh=_mesh,
    compiler_params=_cparams,
    scratch_types=[
        pltpu.VMEM((2, BLK, CH), jnp.int32),
        pltpu.VMEM((2, BLK, CH), jnp.int32),
        pltpu.VMEM((CH, H), jnp.float32),
        pltpu.VMEM((CH, H), jnp.float32),
        pltpu.VMEM((CH, H), jnp.float32),
        pltpu.VMEM((CH, H), jnp.float32),
        pltpu.VMEM((CH,), jnp.float32),
        pltpu.SemaphoreType.DMA,
        pltpu.SemaphoreType.DMA,
        pltpu.SemaphoreType.DMA,
        pltpu.SemaphoreType.DMA,
    ],
)
def _score_kernel(h2_hbm, src2d_hbm, dst2d_hbm, out_hbm,
                  sidxb, didxb, arows0, arows1, brows0, brows1, sbuf,
                  sema0, sema1, semb0, semb1):
    c = lax.axis_index("c")
    s = lax.axis_index("s")
    wid = s * NC + c
    start, cnt = _worker_blocks(wid)
    lanes = lax.iota(jnp.int32, L)
    rots = [(lanes + r) % L for r in range(L)]
    arows = (arows0, arows1)
    brows = (brows0, brows1)
    sema = (sema0, sema1)
    semb = (semb0, semb1)

    def fire(ibx, rx, b):
        pltpu.async_copy(h2_hbm.at[sidxb.at[ibx, rx]], arows[b], sema[b])
        pltpu.async_copy(h2_hbm.at[didxb.at[ibx, rx]], brows[b], semb[b])

    def drain(b):
        pltpu.make_async_copy(h2_hbm.at[sidxb.at[0, 0]], arows[b],
                              sema[b]).wait()
        pltpu.make_async_copy(h2_hbm.at[didxb.at[0, 0]], brows[b],
                              semb[b]).wait()

    pltpu.sync_copy(src2d_hbm.at[pl.ds(start * BLK, BLK)], sidxb.at[0])
    pltpu.sync_copy(dst2d_hbm.at[pl.ds(start * BLK, BLK)], didxb.at[0])
    fire(0, 0, 0)

    def compute_chunk(b, out_base):
        def group_body(g, _):
            eids = lanes + g * L

            def fblk_body(k, acc):
                fbase = k * L
                for j in range(L):
                    colv = rots[j] + fbase
                    a = plsc.load_gather(arows[b], [eids, colv])
                    bb = plsc.load_gather(brows[b], [eids, colv])
                    acc = acc + a * bb
                return acc

            acc = lax.fori_loop(
                0, H // L, fblk_body, jnp.zeros((L,), dtype=jnp.float32))
            sbuf[pl.ds(g * L, L)] = acc
            return 0

        lax.fori_loop(0, CH // L, group_body, 0)
        pltpu.sync_copy(sbuf, out_hbm.at[pl.ds(out_base, CH)])

    def blk_body(p, _):
        ib = p % 2

        @pl.when(p + 1 < cnt)
        def _():
            pltpu.sync_copy(
                src2d_hbm.at[pl.ds((start + p + 1) * BLK, BLK)],
                sidxb.at[1 - ib],
            )
            pltpu.sync_copy(
                dst2d_hbm.at[pl.ds((start + p + 1) * BLK, BLK)],
                didxb.at[1 - ib],
            )

        for r in range(BLK):
            b = r % 2
            nb = 1 - b
            if r < BLK - 1:
                fire(ib, r + 1, nb)
            else:
                @pl.when(p + 1 < cnt)
                def _():
                    fire(1 - ib, 0, nb)
            drain(b)
            compute_chunk(b, ((start + p) * BLK + r) * CH)
        return 0

    lax.fori_loop(0, cnt, blk_body, 0)


def _minmax_body(x_ref, mn_ref, mx_ref):
    i = pl.program_id(0)
    x = x_ref[...]
    bmn = jnp.min(x, axis=0, keepdims=True)
    bmx = jnp.max(x, axis=0, keepdims=True)

    @pl.when(i == 0)
    def _():
        mn_ref[...] = bmn
        mx_ref[...] = bmx

    @pl.when(i > 0)
    def _():
        mn_ref[...] = jnp.minimum(mn_ref[...], bmn)
        mx_ref[...] = jnp.maximum(mx_ref[...], bmx)


def _dense1_body(x_ref, mn_ref, mx_ref, degc_ref, w1_ref, hs1_ref, dinv_ref):
    mn = mn_ref[...]
    mx = mx_ref[...]
    den = jnp.where(mx > mn, mx - mn, jnp.ones_like(mx))
    xf = (x_ref[...] - mn) / den
    dinv = lax.rsqrt(degc_ref[...] + 1.0)
    dinv_ref[...] = dinv
    h = lax.dot_general(xf, w1_ref[...], (((1,), (1,)), ((), ())),
                        preferred_element_type=jnp.float32)
    hs1_ref[...] = h * dinv


def _dense2_body(acc_ref, hs1_ref, dinv_ref, b1_ref, w2_ref, hs2_ref):
    dinv = dinv_ref[...]
    agg = acc_ref[0] + acc_ref[1] + hs1_ref[...]
    h1 = jnp.maximum(agg * dinv + b1_ref[...], 0.0)
    h = lax.dot_general(h1, w2_ref[...], (((1,), (1,)), ((), ())),
                        preferred_element_type=jnp.float32)
    hs2_ref[...] = h * dinv


def _dense3_body(acc_ref, hs2_ref, dinv_ref, b2_ref, h2_ref):
    agg = acc_ref[0] + acc_ref[1] + hs2_ref[...]
    h2_ref[...] = agg * dinv_ref[...] + b2_ref[...]


_RB = 2000
_G = N // _RB
_RBP = 1264
_GP = NPAD // _RBP


def kernel(x, edge_index, W1, b1, W2, b2):
    src = edge_index[0]
    dst = edge_index[1]
    srcp = jnp.pad(src, (0, E_PAD - E)).reshape(NCHUNK_PAD, CH)
    dstp = jnp.pad(dst, (0, E_PAD - E), constant_values=N).reshape(
        NCHUNK_PAD, CH)
    zeros1 = jnp.zeros((DEG_STRIPE,), dtype=jnp.float32)
    zeros2 = jnp.zeros((ROWS_PER_TILE, H), dtype=jnp.float32)

    degp = _deg_kernel(dstp, zeros1).reshape(NC, DEGP)
    degc = jnp.pad((degp[0, :N] + degp[1, :N]).reshape(N, 1),
                   ((0, NPAD - N), (0, 0)))
    xp = jnp.pad(x, ((0, NPAD - N), (0, 0)))

    mn, mx = pl.pallas_call(
        _minmax_body,
        grid=(_G,),
        in_specs=[pl.BlockSpec((_RB, D), lambda i: (i, 0))],
        out_specs=[pl.BlockSpec((1, D), lambda i: (0, 0)),
                   pl.BlockSpec((1, D), lambda i: (0, 0))],
        out_shape=[jax.ShapeDtypeStruct((1, D), jnp.float32),
                   jax.ShapeDtypeStruct((1, D), jnp.float32)],
    )(x)

    hs1, dinv = pl.pallas_call(
        _dense1_body,
        grid=(_GP,),
        in_specs=[
            pl.BlockSpec((_RBP, D), lambda i: (i, 0)),
            pl.BlockSpec((1, D), lambda i: (0, 0)),
            pl.BlockSpec((1, D), lambda i: (0, 0)),
            pl.BlockSpec((_RBP, 1), lambda i: (i, 0)),
            pl.BlockSpec((H, D), lambda i: (0, 0)),
        ],
        out_specs=[pl.BlockSpec((_RBP, H), lambda i: (i, 0)),
                   pl.BlockSpec((_RBP, 1), lambda i: (i, 0))],
        out_shape=[jax.ShapeDtypeStruct((NPAD, H), jnp.float32),
                   jax.ShapeDtypeStruct((NPAD, 1), jnp.float32)],
    )(xp, mn, mx, degc, W1)

    acc1 = _agg_kernel(hs1, srcp, dstp, zeros2)

    hs2 = pl.pallas_call(
        _dense2_body,
        grid=(_GP,),
        in_specs=[
            pl.BlockSpec((NC, _RBP, H), lambda i: (0, i, 0)),
            pl.BlockSpec((_RBP, H), lambda i: (i, 0)),
            pl.BlockSpec((_RBP, 1), lambda i: (i, 0)),
            pl.BlockSpec((H,), lambda i: (0,)),
            pl.BlockSpec((H, H), lambda i: (0, 0)),
        ],
        out_specs=pl.BlockSpec((_RBP, H), lambda i: (i, 0)),
        out_shape=jax.ShapeDtypeStruct((NPAD, H), jnp.float32),
    )(acc1, hs1, dinv, b1, W2)

    acc2 = _agg_kernel(hs2, srcp, dstp, zeros2)

    h2 = pl.pallas_call(
        _dense3_body,
        grid=(_GP,),
        in_specs=[
            pl.BlockSpec((NC, _RBP, H), lambda i: (0, i, 0)),
            pl.BlockSpec((_RBP, H), lambda i: (i, 0)),
            pl.BlockSpec((_RBP, 1), lambda i: (i, 0)),
            pl.BlockSpec((H,), lambda i: (0,)),
        ],
        out_specs=pl.BlockSpec((_RBP, H), lambda i: (i, 0)),
        out_shape=jax.ShapeDtypeStruct((NPAD, H), jnp.float32),
    )(acc2, hs2, dinv, b2)

    return _score_kernel(h2, srcp, dstp)[:E]
